# baseline (speedup 1.0000x reference)
import functools

import jax
import jax.numpy as jnp
from jax import lax
from jax.experimental import pallas as pl
from jax.experimental.pallas import tpu as pltpu
from jax.experimental.pallas import tpu_sc as plsc

V = 1000000
D = 100
H = 100
NTAGS = 50
B = 1024
S = 50

HP = 128
GP = 4 * HP
KP = HP + D
BB = 256
NTOK = B * S

NC = 2
NS = 16
NW = NC * NS
PER_W = NTOK // NW
CH = 64
NCH = PER_W // CH


def _gather_sc(emb, idx3):
    mesh = plsc.VectorSubcoreMesh(core_axis_name="c", subcore_axis_name="s")

    @functools.partial(
        pl.kernel,
        mesh=mesh,
        out_type=jax.ShapeDtypeStruct((NTOK, D), jnp.float32),
        scratch_types=[
            pltpu.VMEM((PER_W,), jnp.int32),
            pltpu.SemaphoreType.DMA((4,)),
        ],
    )
    def gk(emb_hbm, idx_hbm, out_hbm, idx_v, sems):
        wid = lax.axis_index("s") * NC + lax.axis_index("c")
        base = wid * PER_W
        pltpu.sync_copy(idx_hbm.at[wid], idx_v)
        NSEM = 4
        GO = PER_W // (16 * NSEM)

        def wait_group(q):
            for _j in range(16):
                pltpu.make_async_copy(
                    emb_hbm.at[pl.ds(0, 1)], out_hbm.at[pl.ds(base, 1)],
                    sems.at[q],
                ).wait()

        def body(go, carry):
            for q in range(NSEM):
                g = go * NSEM + q

                @pl.when(go >= 1)
                def _():
                    wait_group(q)

                v = idx_v[pl.ds(g * 16, 16)]
                for j in range(16):
                    pltpu.make_async_copy(
                        emb_hbm.at[pl.ds(v[j], 1)],
                        out_hbm.at[pl.ds(base + g * 16 + j, 1)],
                        sems.at[q],
                    ).start()
            return carry

        lax.fori_loop(0, GO, body, 0)
        for q in range(NSEM):
            wait_group(q)

    return gk(emb, idx3)


def _bilstm_head_tc(wv_sbd, Wf, bf, Wb, bb, Wh1, Wh2, bh):
    def kern(wv_ref, Wf_ref, bf_ref, Wb_ref, bb_ref, Wh1_ref, Wh2_ref, bh_ref,
             out_ref, hf_all, hb_all):
        def step(k, carry):
            h_f, c_f, h_b, c_b = carry
            x_f = wv_ref[k]
            x_b = wv_ref[S - 1 - k]
            g_f = jnp.dot(jnp.concatenate([h_f, x_f], axis=1), Wf_ref[...],
                          preferred_element_type=jnp.float32) + bf_ref[...]
            g_b = jnp.dot(jnp.concatenate([h_b, x_b], axis=1), Wb_ref[...],
                          preferred_element_type=jnp.float32) + bb_ref[...]
            i_f = jax.nn.sigmoid(g_f[:, 0:HP])
            f_f = jax.nn.sigmoid(g_f[:, HP:2 * HP])
            gg_f = jnp.tanh(g_f[:, 2 * HP:3 * HP])
            o_f = jax.nn.sigmoid(g_f[:, 3 * HP:4 * HP])
            c_f = f_f * c_f + i_f * gg_f
            h_f = o_f * jnp.tanh(c_f)
            i_b = jax.nn.sigmoid(g_b[:, 0:HP])
            f_b = jax.nn.sigmoid(g_b[:, HP:2 * HP])
            gg_b = jnp.tanh(g_b[:, 2 * HP:3 * HP])
            o_b = jax.nn.sigmoid(g_b[:, 3 * HP:4 * HP])
            c_b = f_b * c_b + i_b * gg_b
            h_b = o_b * jnp.tanh(c_b)
            hf_all[k] = h_f
            hb_all[S - 1 - k] = h_b
            return (h_f, c_f, h_b, c_b)

        z = jnp.zeros((BB, HP), jnp.float32)
        lax.fori_loop(0, S, step, (z, z, z, z))
        hf = hf_all[...].reshape(S * BB, HP)
        hb = hb_all[...].reshape(S * BB, HP)
        preds = (jnp.dot(hf, Wh1_ref[...], preferred_element_type=jnp.float32)
                 + jnp.dot(hb, Wh2_ref[...], preferred_element_type=jnp.float32)
                 + bh_ref[...])
        out_ref[...] = preds.reshape(S, BB, NTAGS)

    return pl.pallas_call(
        kern,
        grid=(B // BB,),
        in_specs=[
            pl.BlockSpec((S, BB, D), lambda i: (0, i, 0)),
            pl.BlockSpec((KP, GP), lambda i: (0, 0)),
            pl.BlockSpec((1, GP), lambda i: (0, 0)),
            pl.BlockSpec((KP, GP), lambda i: (0, 0)),
            pl.BlockSpec((1, GP), lambda i: (0, 0)),
            pl.BlockSpec((HP, NTAGS), lambda i: (0, 0)),
            pl.BlockSpec((HP, NTAGS), lambda i: (0, 0)),
            pl.BlockSpec((1, NTAGS), lambda i: (0, 0)),
        ],
        out_specs=pl.BlockSpec((S, BB, NTAGS), lambda i: (0, i, 0)),
        out_shape=jax.ShapeDtypeStruct((S, B, NTAGS), jnp.float32),
        scratch_shapes=[
            pltpu.VMEM((S, BB, HP), jnp.float32),
            pltpu.VMEM((S, BB, HP), jnp.float32),
        ],
    )(wv_sbd, Wf, bf, Wb, bb, Wh1, Wh2, bh)


def _pack_gates(W):
    X = W.shape[1]
    Wt = W.T.reshape(X, 4, H)
    Wt = jnp.pad(Wt, ((0, 0), (0, 0), (0, HP - H)))
    return Wt.reshape(X, GP)


def _pack_dir(W_ih, W_hh, b_ih, b_hh):
    Whh = jnp.pad(_pack_gates(W_hh), ((0, HP - H), (0, 0)))
    Wih = _pack_gates(W_ih)
    Wcat = jnp.concatenate([Whh, Wih], axis=0)
    bias = jnp.pad((b_ih + b_hh).reshape(4, H),
                   ((0, 0), (0, HP - H))).reshape(1, GP)
    return Wcat, bias


def kernel(input_data, emb, W_ih_f, W_hh_f, b_ih_f, b_hh_f,
           W_ih_b, W_hh_b, b_ih_b, b_hh_b, W_head, b_head):
    idx3 = input_data.astype(jnp.int32).T.reshape(NW, PER_W)
    wv = _gather_sc(emb, idx3).reshape(S, B, D)

    Wf, bf = _pack_dir(W_ih_f, W_hh_f, b_ih_f, b_hh_f)
    Wb, bb = _pack_dir(W_ih_b, W_hh_b, b_ih_b, b_hh_b)
    Wh1 = jnp.pad(W_head[:, :H].T, ((0, HP - H), (0, 0)))
    Wh2 = jnp.pad(W_head[:, H:].T, ((0, HP - H), (0, 0)))
    bh = b_head.reshape(1, NTAGS)

    out = _bilstm_head_tc(wv, Wf, bf, Wb, bb, Wh1, Wh2, bh)
    return jnp.swapaxes(out, 0, 1)


# speedup vs baseline: 2.2531x; 2.2531x over previous
import functools

import jax
import jax.numpy as jnp
from jax import lax
from jax.experimental import pallas as pl
from jax.experimental.pallas import tpu as pltpu
from jax.experimental.pallas import tpu_sc as plsc

V = 1000000
D = 100
H = 100
NTAGS = 50
B = 1024
S = 50

HP = 128
GP = 4 * HP
KP = HP + D
BB = 256
NTOK = B * S

NC = 2
NS = 16
NW = NC * NS
PER_W = NTOK // NW
CH = 64
NCH = PER_W // CH


def _gather_sc(emb, idx3):
    mesh = plsc.VectorSubcoreMesh(core_axis_name="c", subcore_axis_name="s")

    @functools.partial(
        pl.kernel,
        mesh=mesh,
        out_type=jax.ShapeDtypeStruct((NTOK, D), jnp.float32),
        scratch_types=[
            pltpu.VMEM((PER_W,), jnp.int32),
            pltpu.VMEM((2, 400, D), jnp.float32),
            pltpu.SemaphoreType.DMA,
            pltpu.SemaphoreType.DMA((2,)),
        ],
    )
    def gk(emb_hbm, idx_hbm, out_hbm, idx_v, bufs, gsem, wsems):
        wid = lax.axis_index("s") * NC + lax.axis_index("c")
        base = wid * PER_W
        pltpu.sync_copy(idx_hbm.at[wid], idx_v)
        CHK = 400
        NCHK = PER_W // CHK
        GPC = CHK // 16

        def wait_gathers(n):
            def w(i, carry):
                pltpu.make_async_copy(
                    emb_hbm.at[pl.ds(0, 1)], bufs.at[0, pl.ds(0, 1)], gsem
                ).wait()
                return carry
            lax.fori_loop(0, n, w, 0)

        for k in range(NCHK):
            b = k % 2
            if k >= 2:
                pltpu.make_async_copy(
                    bufs.at[b], out_hbm.at[pl.ds(base, CHK)], wsems.at[b]
                ).wait()

            def body(g, carry, k=k, b=b):
                v = idx_v[pl.ds(k * CHK + g * 16, 16)]
                for j in range(16):
                    pltpu.make_async_copy(
                        emb_hbm.at[pl.ds(v[j], 1)],
                        bufs.at[b, pl.ds(g * 16 + j, 1)],
                        gsem,
                    ).start()

                @pl.when(g >= 1)
                def _():
                    wait_gathers(16)

                return carry

            lax.fori_loop(0, GPC, body, 0)
            wait_gathers(16)
            pltpu.make_async_copy(
                bufs.at[b], out_hbm.at[pl.ds(base + k * CHK, CHK)], wsems.at[b]
            ).start()

        for b in range(2):
            pltpu.make_async_copy(
                bufs.at[b], out_hbm.at[pl.ds(base, CHK)], wsems.at[b]
            ).wait()

    return gk(emb, idx3)


def _bilstm_head_tc(wv_sbd, Wf, bf, Wb, bb, Wh1, Wh2, bh):
    def kern(wv_ref, Wf_ref, bf_ref, Wb_ref, bb_ref, Wh1_ref, Wh2_ref, bh_ref,
             out_ref, hf_all, hb_all):
        def step(k, carry):
            h_f, c_f, h_b, c_b = carry
            x_f = wv_ref[k]
            x_b = wv_ref[S - 1 - k]
            g_f = jnp.dot(jnp.concatenate([h_f, x_f], axis=1), Wf_ref[...],
                          preferred_element_type=jnp.float32) + bf_ref[...]
            g_b = jnp.dot(jnp.concatenate([h_b, x_b], axis=1), Wb_ref[...],
                          preferred_element_type=jnp.float32) + bb_ref[...]
            i_f = jax.nn.sigmoid(g_f[:, 0:HP])
            f_f = jax.nn.sigmoid(g_f[:, HP:2 * HP])
            gg_f = jnp.tanh(g_f[:, 2 * HP:3 * HP])
            o_f = jax.nn.sigmoid(g_f[:, 3 * HP:4 * HP])
            c_f = f_f * c_f + i_f * gg_f
            h_f = o_f * jnp.tanh(c_f)
            i_b = jax.nn.sigmoid(g_b[:, 0:HP])
            f_b = jax.nn.sigmoid(g_b[:, HP:2 * HP])
            gg_b = jnp.tanh(g_b[:, 2 * HP:3 * HP])
            o_b = jax.nn.sigmoid(g_b[:, 3 * HP:4 * HP])
            c_b = f_b * c_b + i_b * gg_b
            h_b = o_b * jnp.tanh(c_b)
            hf_all[k] = h_f
            hb_all[S - 1 - k] = h_b
            return (h_f, c_f, h_b, c_b)

        z = jnp.zeros((BB, HP), jnp.float32)
        lax.fori_loop(0, S, step, (z, z, z, z))
        hf = hf_all[...].reshape(S * BB, HP)
        hb = hb_all[...].reshape(S * BB, HP)
        preds = (jnp.dot(hf, Wh1_ref[...], preferred_element_type=jnp.float32)
                 + jnp.dot(hb, Wh2_ref[...], preferred_element_type=jnp.float32)
                 + bh_ref[...])
        out_ref[...] = preds.reshape(S, BB, NTAGS)

    return pl.pallas_call(
        kern,
        grid=(B // BB,),
        in_specs=[
            pl.BlockSpec((S, BB, D), lambda i: (0, i, 0)),
            pl.BlockSpec((KP, GP), lambda i: (0, 0)),
            pl.BlockSpec((1, GP), lambda i: (0, 0)),
            pl.BlockSpec((KP, GP), lambda i: (0, 0)),
            pl.BlockSpec((1, GP), lambda i: (0, 0)),
            pl.BlockSpec((HP, NTAGS), lambda i: (0, 0)),
            pl.BlockSpec((HP, NTAGS), lambda i: (0, 0)),
            pl.BlockSpec((1, NTAGS), lambda i: (0, 0)),
        ],
        out_specs=pl.BlockSpec((S, BB, NTAGS), lambda i: (0, i, 0)),
        out_shape=jax.ShapeDtypeStruct((S, B, NTAGS), jnp.float32),
        scratch_shapes=[
            pltpu.VMEM((S, BB, HP), jnp.float32),
            pltpu.VMEM((S, BB, HP), jnp.float32),
        ],
    )(wv_sbd, Wf, bf, Wb, bb, Wh1, Wh2, bh)


def _pack_gates(W):
    X = W.shape[1]
    Wt = W.T.reshape(X, 4, H)
    Wt = jnp.pad(Wt, ((0, 0), (0, 0), (0, HP - H)))
    return Wt.reshape(X, GP)


def _pack_dir(W_ih, W_hh, b_ih, b_hh):
    Whh = jnp.pad(_pack_gates(W_hh), ((0, HP - H), (0, 0)))
    Wih = _pack_gates(W_ih)
    Wcat = jnp.concatenate([Whh, Wih], axis=0)
    bias = jnp.pad((b_ih + b_hh).reshape(4, H),
                   ((0, 0), (0, HP - H))).reshape(1, GP)
    return Wcat, bias


def kernel(input_data, emb, W_ih_f, W_hh_f, b_ih_f, b_hh_f,
           W_ih_b, W_hh_b, b_ih_b, b_hh_b, W_head, b_head):
    idx3 = input_data.astype(jnp.int32).T.reshape(NW, PER_W)
    wv = _gather_sc(emb, idx3).reshape(S, B, D)

    Wf, bf = _pack_dir(W_ih_f, W_hh_f, b_ih_f, b_hh_f)
    Wb, bb = _pack_dir(W_ih_b, W_hh_b, b_ih_b, b_hh_b)
    Wh1 = jnp.pad(W_head[:, :H].T, ((0, HP - H), (0, 0)))
    Wh2 = jnp.pad(W_head[:, H:].T, ((0, HP - H), (0, 0)))
    bh = b_head.reshape(1, NTAGS)

    out = _bilstm_head_tc(wv, Wf, bf, Wb, bb, Wh1, Wh2, bh)
    return jnp.swapaxes(out, 0, 1)
